# baseline (speedup 1.0000x reference)
import functools

import jax
import jax.numpy as jnp
from jax import lax
from jax.experimental import pallas as pl
from jax.experimental.pallas import tpu as pltpu
from jax.experimental.pallas import tpu_sc as plsc

D_IN = 1024
N_DICT = 32768
K_TOP = 32
N_TOK = 2048

TN = 2048
NGT = 128
G_MEM = TN // NGT
N_GRP = (N_DICT // TN) * NGT
N_CAND = K_TOP * G_MEM

N_WORKERS = 32
RPW = N_TOK // N_WORKERS

@functools.cache
def _sc_mesh():
    return plsc.VectorSubcoreMesh(core_axis_name="c", subcore_axis_name="s")


def _encode_body(x_ref, w_ref, be_ref, bd_ref, pre_ref, gmax_ref):
    xa = x_ref[...] - bd_ref[...]
    acts = lax.dot_general(xa, w_ref[...], (((1,), (1,)), ((), ())),
                           preferred_element_type=jnp.float32)
    acts = jnp.maximum(acts + be_ref[...], 0.0)
    pre_ref[...] = acts
    gm = acts[:, 0:NGT]
    for m in range(1, G_MEM):
        gm = jnp.maximum(gm, acts[:, m * NGT:(m + 1) * NGT])
    gmax_ref[...] = gm


_TM1 = 512


def _encode(x, w_enc, be2, bd2, n_tok):
    return pl.pallas_call(
        _encode_body,
        grid=(N_DICT // TN, n_tok // _TM1),
        in_specs=[
            pl.BlockSpec((_TM1, D_IN), lambda d, t: (t, 0)),
            pl.BlockSpec((TN, D_IN), lambda d, t: (d, 0)),
            pl.BlockSpec((1, TN), lambda d, t: (0, d)),
            pl.BlockSpec((1, D_IN), lambda d, t: (0, 0)),
        ],
        out_specs=[
            pl.BlockSpec((_TM1, TN), lambda d, t: (t, d)),
            pl.BlockSpec((_TM1, NGT), lambda d, t: (t, d)),
        ],
        out_shape=[
            jax.ShapeDtypeStruct((n_tok, N_DICT), jnp.float32),
            jax.ShapeDtypeStruct((n_tok, N_GRP), jnp.float32),
        ],
    )(x, w_enc, be2, bd2)


_TM2 = 256


def _gsel_body(gmax_ref, gsel_ref):
    g = gmax_ref[...]
    cols = lax.broadcasted_iota(jnp.int32, (_TM2, N_GRP), 1).astype(jnp.float32)
    picks = []
    for _ in range(K_TOP):
        m = jnp.max(g, axis=1, keepdims=True)
        sel = g >= m
        c = jnp.min(jnp.where(sel, cols, 3e9), axis=1, keepdims=True)
        picks.append(c)
        g = jnp.where(cols == c, -1.0, g)
    gsel_ref[...] = jnp.concatenate(picks, axis=1).astype(jnp.int32)


def _gsel(gmax, n_tok):
    return pl.pallas_call(
        _gsel_body,
        grid=(n_tok // _TM2,),
        in_specs=[pl.BlockSpec((_TM2, N_GRP), lambda t: (t, 0))],
        out_specs=pl.BlockSpec((_TM2, K_TOP), lambda t: (t, 0)),
        out_shape=jax.ShapeDtypeStruct((n_tok, K_TOP), jnp.int32),
    )(gmax)


def _make_cand_gather_kernel(rpw):
  def _cand_gather_kernel(gsel2_hbm, pre_hbm, vals_hbm, cols_hbm,
                          gsel_v, idx_v, cols_v, vals_v, sem):
    wid = lax.axis_index("s") * 2 + lax.axis_index("c")
    r0 = wid * rpw
    pltpu.sync_copy(gsel2_hbm.at[pl.ds(r0 * 2, rpw * 2)], gsel_v)

    def row_body(r, _):
        row = r0 + r
        g0 = gsel_v[2 * r]
        g1 = gsel_v[2 * r + 1]
        c0 = (g0 >> 7) * TN + (g0 & (NGT - 1))
        c1 = (g1 >> 7) * TN + (g1 & (NGT - 1))
        tbase = (row >> 3) * (N_DICT // 128) * 1024 + (row & 7) * 128
        e0 = tbase + (g0 >> 7) * 16 * 1024 + (g0 & (NGT - 1))
        e1 = tbase + (g1 >> 7) * 16 * 1024 + (g1 & (NGT - 1))
        for m in range(G_MEM):
            j = m // 4
            lane = (m % 4) * 32
            idx_v[4 * r + j, pl.ds(lane, 16)] = e0 + 1024 * m
            idx_v[4 * r + j, pl.ds(lane + 16, 16)] = e1 + 1024 * m
            cols_v[4 * r + j, pl.ds(lane, 16)] = c0 + 128 * m
            cols_v[4 * r + j, pl.ds(lane + 16, 16)] = c1 + 128 * m
        for j in range(4):
            pltpu.async_copy(pre_hbm.at[idx_v.at[4 * r + j]],
                             vals_v.at[4 * r + j], sem)
        return ()

    lax.fori_loop(0, rpw, row_body, (), unroll=False)
    pltpu.make_async_copy(vals_hbm.at[pl.ds(r0 * 4, rpw * 4)], vals_v, sem).wait()
    pltpu.sync_copy(vals_v, vals_hbm.at[pl.ds(r0 * 4, rpw * 4)])
    pltpu.sync_copy(cols_v, cols_hbm.at[pl.ds(r0 * 4, rpw * 4)])
  return _cand_gather_kernel


def _cand_gather(gsel, pre, n_tok):
    rpw = n_tok // N_WORKERS
    gsel2 = gsel.reshape(n_tok * 2, 16)
    pre_flat = (pre.reshape(n_tok // 8, 8, N_DICT // 128, 128)
                .transpose(0, 2, 1, 3).reshape(n_tok * N_DICT))
    out = pl.kernel(
        _make_cand_gather_kernel(rpw),
        out_type=(
            jax.ShapeDtypeStruct((n_tok * 4, 128), jnp.float32),
            jax.ShapeDtypeStruct((n_tok * 4, 128), jnp.int32),
        ),
        mesh=_sc_mesh(),
        scratch_types=[
            pltpu.VMEM((rpw * 2, 16), jnp.int32),
            pltpu.VMEM((rpw * 4, 128), jnp.int32),
            pltpu.VMEM((rpw * 4, 128), jnp.int32),
            pltpu.VMEM((rpw * 4, 128), jnp.float32),
            pltpu.SemaphoreType.DMA,
        ],
    )(gsel2, pre_flat)
    vals4, cols4 = out
    return vals4.reshape(n_tok, N_CAND), cols4.reshape(n_tok, N_CAND)


def _topk_body(vals_ref, cols_ref, acts_ref, idx_ref):
    v = vals_ref[...]
    cols = cols_ref[...].astype(jnp.float32)
    out_v, out_c = [], []
    for _ in range(K_TOP):
        m = jnp.max(v, axis=1, keepdims=True)
        sel = v >= m
        c = jnp.min(jnp.where(sel, cols, 3e9), axis=1, keepdims=True)
        out_v.append(m)
        out_c.append(c)
        v = jnp.where(cols == c, -1.0, v)
    acts_ref[...] = jnp.concatenate(out_v, axis=1)
    idx_ref[...] = jnp.concatenate(out_c, axis=1).astype(jnp.int32)


def _topk(cand_vals, cand_cols, n_tok):
    return pl.pallas_call(
        _topk_body,
        grid=(n_tok // _TM2,),
        in_specs=[
            pl.BlockSpec((_TM2, N_CAND), lambda t: (t, 0)),
            pl.BlockSpec((_TM2, N_CAND), lambda t: (t, 0)),
        ],
        out_specs=[
            pl.BlockSpec((_TM2, K_TOP), lambda t: (t, 0)),
            pl.BlockSpec((_TM2, K_TOP), lambda t: (t, 0)),
        ],
        out_shape=[
            jax.ShapeDtypeStruct((n_tok, K_TOP), jnp.float32),
            jax.ShapeDtypeStruct((n_tok, K_TOP), jnp.int32),
        ],
    )(cand_vals, cand_cols)


def _make_decode_kernel(rpw):
  def _decode_kernel(idx_hbm, acts_hbm, wdec_hbm, bdec_hbm, sae_hbm,
                     idx_v, acts_v, rows_v, out_v, bdec_v, sem0, sem1):
    wid = lax.axis_index("s") * 2 + lax.axis_index("c")
    r0 = wid * rpw
    pltpu.sync_copy(idx_hbm.at[pl.ds(r0, rpw)], idx_v)
    pltpu.sync_copy(acts_hbm.at[pl.ds(r0 * K_TOP, rpw * K_TOP)], acts_v)
    pltpu.sync_copy(bdec_hbm, bdec_v)
    sems = (sem0, sem1)

    def fire(row, buf, sem):
        pltpu.async_copy(wdec_hbm.at[idx_v.at[row]], rows_v.at[buf], sem)

    def drain(buf, sem):
        pltpu.make_async_copy(wdec_hbm.at[pl.ds(0, K_TOP)],
                              rows_v.at[buf], sem).wait()

    def _splat(vec, lane):
        idx = jnp.full((16,), lane, jnp.int32)
        return lax.gather(
            vec, idx[:, None],
            dimension_numbers=lax.GatherDimensionNumbers(
                offset_dims=(), collapsed_slice_dims=(0,),
                start_index_map=(0,)),
            slice_sizes=(1,),
            mode=lax.GatherScatterMode.PROMISE_IN_BOUNDS)

    def compute_row(r, buf):
        a_lo = acts_v[pl.ds(r * K_TOP, 16)]
        a_hi = acts_v[pl.ds(r * K_TOP + 16, 16)]
        splats = [_splat(a_lo, k) for k in range(16)]
        splats += [_splat(a_hi, k) for k in range(16)]

        def chunk_body(c, _):
            base = c * 256
            acc = [jnp.zeros((16,), jnp.float32) for _ in range(16)]
            for k in range(K_TOP):
                s = splats[k]
                for v in range(16):
                    acc[v] = acc[v] + s * rows_v[buf, k, pl.ds(base + v * 16, 16)]
            for v in range(16):
                out_v[buf, pl.ds(base + v * 16, 16)] = (
                    acc[v] + bdec_v[pl.ds(base + v * 16, 16)])
            return ()

        lax.fori_loop(0, D_IN // 256, chunk_body, (), unroll=False)
        pltpu.sync_copy(out_v.at[buf], sae_hbm.at[r0 + r])

    fire(0, 0, sems[0])

    def pair_body(i, _):
        a = 2 * i
        fire(a + 1, 1, sems[1])
        drain(0, sems[0])
        compute_row(a, 0)
        fire(jnp.minimum(a + 2, rpw - 1), 0, sems[0])
        drain(1, sems[1])
        compute_row(a + 1, 1)
        return ()

    lax.fori_loop(0, rpw // 2, pair_body, (), unroll=False)
    drain(0, sems[0])
  return _decode_kernel


def _decode(top_idx, top_acts, w_dec, b_dec, n_tok):
    rpw = n_tok // N_WORKERS
    return pl.kernel(
        _make_decode_kernel(rpw),
        out_type=jax.ShapeDtypeStruct((n_tok, D_IN), jnp.float32),
        mesh=_sc_mesh(),
        scratch_types=[
            pltpu.VMEM((rpw, K_TOP), jnp.int32),
            pltpu.VMEM((rpw * K_TOP,), jnp.float32),
            pltpu.VMEM((2, K_TOP, D_IN), jnp.float32),
            pltpu.VMEM((2, D_IN), jnp.float32),
            pltpu.VMEM((D_IN,), jnp.float32),
            pltpu.SemaphoreType.DMA,
            pltpu.SemaphoreType.DMA,
        ],
    )(top_idx, top_acts.reshape(n_tok * K_TOP), w_dec, b_dec)


def _fvu_body(x_ref, sae_ref, fvu_ref):
    x = x_ref[...]
    e = sae_ref[...] - x
    l2 = jnp.sum(e * e)
    mu = jnp.mean(x, axis=0, keepdims=True)
    d = x - mu
    tv = jnp.sum(d * d)
    tv = jnp.where(tv == 0.0, 1.0, tv)
    fvu_ref[...] = jnp.reshape(l2 / tv, (1, 1))


def _fvu(x, sae):
    return pl.pallas_call(
        _fvu_body,
        out_shape=jax.ShapeDtypeStruct((1, 1), jnp.float32),
    )(x, sae)


def kernel(x, W_enc, b_enc, W_dec, b_dec):
    be2 = b_enc.reshape(1, N_DICT)
    bd2 = b_dec.reshape(1, D_IN)
    splits = (1536, 512)
    parts = []
    start = 0
    for n_h in splits:
        xh = lax.slice_in_dim(x, start, start + n_h, axis=0)
        start += n_h
        pre, gmax = _encode(xh, W_enc, be2, bd2, n_h)
        gsel = _gsel(gmax, n_h)
        cand_vals, cand_cols = _cand_gather(gsel, pre, n_h)
        top_acts, top_idx = _topk(cand_vals, cand_cols, n_h)
        sae_out = _decode(top_idx, top_acts, W_dec, b_dec, n_h)
        parts.append((sae_out, top_acts, top_idx))
    sae_out = jnp.concatenate([p[0] for p in parts], axis=0)
    top_acts = jnp.concatenate([p[1] for p in parts], axis=0)
    top_idx = jnp.concatenate([p[2] for p in parts], axis=0)
    fvu = _fvu(x, sae_out).reshape(())
    zero = jnp.zeros((), x.dtype)
    return sae_out, top_acts, top_idx, fvu, zero, zero


# speedup vs baseline: 1.1252x; 1.1252x over previous
import functools

import jax
import jax.numpy as jnp
from jax import lax
from jax.experimental import pallas as pl
from jax.experimental.pallas import tpu as pltpu
from jax.experimental.pallas import tpu_sc as plsc

D_IN = 1024
N_DICT = 32768
K_TOP = 32
N_TOK = 2048

TN = 2048
NGT = 128
G_MEM = TN // NGT
N_GRP = (N_DICT // TN) * NGT
N_CAND = K_TOP * G_MEM

N_WORKERS = 32
RPW = N_TOK // N_WORKERS

@functools.cache
def _sc_mesh():
    return plsc.VectorSubcoreMesh(core_axis_name="c", subcore_axis_name="s")


def _encode_body(x_ref, w_ref, be_ref, bd_ref, pre_ref, gmax_ref):
    xa = x_ref[...] - bd_ref[...]
    acts = lax.dot_general(xa, w_ref[...], (((1,), (1,)), ((), ())),
                           preferred_element_type=jnp.float32)
    acts = jnp.maximum(acts + be_ref[...], 0.0)
    pre_ref[...] = acts
    gm = acts[:, 0:NGT]
    for m in range(1, G_MEM):
        gm = jnp.maximum(gm, acts[:, m * NGT:(m + 1) * NGT])
    gmax_ref[...] = gm


_TM1 = 1024


def _encode(x, w_enc, be2, bd2, n_tok):
    return pl.pallas_call(
        _encode_body,
        grid=(N_DICT // TN, n_tok // _TM1),
        in_specs=[
            pl.BlockSpec((_TM1, D_IN), lambda d, t: (t, 0)),
            pl.BlockSpec((TN, D_IN), lambda d, t: (d, 0)),
            pl.BlockSpec((1, TN), lambda d, t: (0, d)),
            pl.BlockSpec((1, D_IN), lambda d, t: (0, 0)),
        ],
        out_specs=[
            pl.BlockSpec((_TM1, TN), lambda d, t: (t, d)),
            pl.BlockSpec((_TM1, NGT), lambda d, t: (t, d)),
        ],
        out_shape=[
            jax.ShapeDtypeStruct((n_tok, N_DICT), jnp.float32),
            jax.ShapeDtypeStruct((n_tok, N_GRP), jnp.float32),
        ],
    )(x, w_enc, be2, bd2)


_TM2 = 256


def _gsel_body(gmax_ref, gsel_ref):
    g = gmax_ref[...]
    cols = lax.broadcasted_iota(jnp.int32, (_TM2, N_GRP), 1).astype(jnp.float32)
    picks = []
    for _ in range(K_TOP):
        m = jnp.max(g, axis=1, keepdims=True)
        sel = g >= m
        c = jnp.min(jnp.where(sel, cols, 3e9), axis=1, keepdims=True)
        picks.append(c)
        g = jnp.where(cols == c, -1.0, g)
    gsel_ref[...] = jnp.concatenate(picks, axis=1).astype(jnp.int32)


def _gsel(gmax, n_tok):
    return pl.pallas_call(
        _gsel_body,
        grid=(n_tok // _TM2,),
        in_specs=[pl.BlockSpec((_TM2, N_GRP), lambda t: (t, 0))],
        out_specs=pl.BlockSpec((_TM2, K_TOP), lambda t: (t, 0)),
        out_shape=jax.ShapeDtypeStruct((n_tok, K_TOP), jnp.int32),
    )(gmax)


def _make_cand_gather_kernel(rpw):
  def _cand_gather_kernel(gsel2_hbm, pre_hbm, vals_hbm, cols_hbm,
                          gsel_v, idx_v, cols_v, vals_v, sem):
    wid = lax.axis_index("s") * 2 + lax.axis_index("c")
    r0 = wid * rpw
    pltpu.sync_copy(gsel2_hbm.at[pl.ds(r0 * 2, rpw * 2)], gsel_v)

    def row_body(r, _):
        row = r0 + r
        g0 = gsel_v[2 * r]
        g1 = gsel_v[2 * r + 1]
        c0 = (g0 >> 7) * TN + (g0 & (NGT - 1))
        c1 = (g1 >> 7) * TN + (g1 & (NGT - 1))
        tbase = (row >> 3) * (N_DICT // 128) * 1024 + (row & 7) * 128
        e0 = tbase + (g0 >> 7) * 16 * 1024 + (g0 & (NGT - 1))
        e1 = tbase + (g1 >> 7) * 16 * 1024 + (g1 & (NGT - 1))
        for m in range(G_MEM):
            j = m // 4
            lane = (m % 4) * 32
            idx_v[4 * r + j, pl.ds(lane, 16)] = e0 + 1024 * m
            idx_v[4 * r + j, pl.ds(lane + 16, 16)] = e1 + 1024 * m
            cols_v[4 * r + j, pl.ds(lane, 16)] = c0 + 128 * m
            cols_v[4 * r + j, pl.ds(lane + 16, 16)] = c1 + 128 * m
        for j in range(4):
            pltpu.async_copy(pre_hbm.at[idx_v.at[4 * r + j]],
                             vals_v.at[4 * r + j], sem)
        return ()

    lax.fori_loop(0, rpw, row_body, (), unroll=False)
    pltpu.make_async_copy(vals_hbm.at[pl.ds(r0 * 4, rpw * 4)], vals_v, sem).wait()
    pltpu.sync_copy(vals_v, vals_hbm.at[pl.ds(r0 * 4, rpw * 4)])
    pltpu.sync_copy(cols_v, cols_hbm.at[pl.ds(r0 * 4, rpw * 4)])
  return _cand_gather_kernel


def _cand_gather(gsel, pre, n_tok):
    rpw = n_tok // N_WORKERS
    gsel2 = gsel.reshape(n_tok * 2, 16)
    pre_flat = (pre.reshape(n_tok // 8, 8, N_DICT // 128, 128)
                .transpose(0, 2, 1, 3).reshape(n_tok * N_DICT))
    out = pl.kernel(
        _make_cand_gather_kernel(rpw),
        out_type=(
            jax.ShapeDtypeStruct((n_tok * 4, 128), jnp.float32),
            jax.ShapeDtypeStruct((n_tok * 4, 128), jnp.int32),
        ),
        mesh=_sc_mesh(),
        scratch_types=[
            pltpu.VMEM((rpw * 2, 16), jnp.int32),
            pltpu.VMEM((rpw * 4, 128), jnp.int32),
            pltpu.VMEM((rpw * 4, 128), jnp.int32),
            pltpu.VMEM((rpw * 4, 128), jnp.float32),
            pltpu.SemaphoreType.DMA,
        ],
    )(gsel2, pre_flat)
    vals4, cols4 = out
    return vals4.reshape(n_tok, N_CAND), cols4.reshape(n_tok, N_CAND)


def _topk_body(vals_ref, cols_ref, acts_ref, idx_ref):
    v = vals_ref[...]
    cols = cols_ref[...].astype(jnp.float32)
    out_v, out_c = [], []
    for _ in range(K_TOP):
        m = jnp.max(v, axis=1, keepdims=True)
        sel = v >= m
        c = jnp.min(jnp.where(sel, cols, 3e9), axis=1, keepdims=True)
        out_v.append(m)
        out_c.append(c)
        v = jnp.where(cols == c, -1.0, v)
    acts_ref[...] = jnp.concatenate(out_v, axis=1)
    idx_ref[...] = jnp.concatenate(out_c, axis=1).astype(jnp.int32)


def _topk(cand_vals, cand_cols, n_tok):
    return pl.pallas_call(
        _topk_body,
        grid=(n_tok // _TM2,),
        in_specs=[
            pl.BlockSpec((_TM2, N_CAND), lambda t: (t, 0)),
            pl.BlockSpec((_TM2, N_CAND), lambda t: (t, 0)),
        ],
        out_specs=[
            pl.BlockSpec((_TM2, K_TOP), lambda t: (t, 0)),
            pl.BlockSpec((_TM2, K_TOP), lambda t: (t, 0)),
        ],
        out_shape=[
            jax.ShapeDtypeStruct((n_tok, K_TOP), jnp.float32),
            jax.ShapeDtypeStruct((n_tok, K_TOP), jnp.int32),
        ],
    )(cand_vals, cand_cols)


def _make_decode_kernel(rpw):
  def _decode_kernel(idx_hbm, acts_hbm, wdec_hbm, bdec_hbm, sae_hbm,
                     idx_v, acts_v, rows_v, out_v, bdec_v, sem0, sem1):
    wid = lax.axis_index("s") * 2 + lax.axis_index("c")
    r0 = wid * rpw
    pltpu.sync_copy(idx_hbm.at[pl.ds(r0, rpw)], idx_v)
    pltpu.sync_copy(acts_hbm.at[pl.ds(r0 * K_TOP, rpw * K_TOP)], acts_v)
    pltpu.sync_copy(bdec_hbm, bdec_v)
    sems = (sem0, sem1)

    def fire(row, buf, sem):
        pltpu.async_copy(wdec_hbm.at[idx_v.at[row]], rows_v.at[buf], sem)

    def drain(buf, sem):
        pltpu.make_async_copy(wdec_hbm.at[pl.ds(0, K_TOP)],
                              rows_v.at[buf], sem).wait()

    def _splat(vec, lane):
        idx = jnp.full((16,), lane, jnp.int32)
        return lax.gather(
            vec, idx[:, None],
            dimension_numbers=lax.GatherDimensionNumbers(
                offset_dims=(), collapsed_slice_dims=(0,),
                start_index_map=(0,)),
            slice_sizes=(1,),
            mode=lax.GatherScatterMode.PROMISE_IN_BOUNDS)

    def compute_row(r, buf):
        a_lo = acts_v[pl.ds(r * K_TOP, 16)]
        a_hi = acts_v[pl.ds(r * K_TOP + 16, 16)]
        splats = [_splat(a_lo, k) for k in range(16)]
        splats += [_splat(a_hi, k) for k in range(16)]

        def chunk_body(c, _):
            base = c * 256
            acc = [jnp.zeros((16,), jnp.float32) for _ in range(16)]
            for k in range(K_TOP):
                s = splats[k]
                for v in range(16):
                    acc[v] = acc[v] + s * rows_v[buf, k, pl.ds(base + v * 16, 16)]
            for v in range(16):
                out_v[buf, pl.ds(base + v * 16, 16)] = (
                    acc[v] + bdec_v[pl.ds(base + v * 16, 16)])
            return ()

        lax.fori_loop(0, D_IN // 256, chunk_body, (), unroll=False)
        pltpu.sync_copy(out_v.at[buf], sae_hbm.at[r0 + r])

    fire(0, 0, sems[0])

    def pair_body(i, _):
        a = 2 * i
        fire(a + 1, 1, sems[1])
        drain(0, sems[0])
        compute_row(a, 0)
        fire(jnp.minimum(a + 2, rpw - 1), 0, sems[0])
        drain(1, sems[1])
        compute_row(a + 1, 1)
        return ()

    lax.fori_loop(0, rpw // 2, pair_body, (), unroll=False)
    drain(0, sems[0])
  return _decode_kernel


def _decode(top_idx, top_acts, w_dec, b_dec, n_tok):
    rpw = n_tok // N_WORKERS
    return pl.kernel(
        _make_decode_kernel(rpw),
        out_type=jax.ShapeDtypeStruct((n_tok, D_IN), jnp.float32),
        mesh=_sc_mesh(),
        scratch_types=[
            pltpu.VMEM((rpw, K_TOP), jnp.int32),
            pltpu.VMEM((rpw * K_TOP,), jnp.float32),
            pltpu.VMEM((2, K_TOP, D_IN), jnp.float32),
            pltpu.VMEM((2, D_IN), jnp.float32),
            pltpu.VMEM((D_IN,), jnp.float32),
            pltpu.SemaphoreType.DMA,
            pltpu.SemaphoreType.DMA,
        ],
    )(top_idx, top_acts.reshape(n_tok * K_TOP), w_dec, b_dec)


def _fvu_body(x_ref, sae_ref, fvu_ref):
    x = x_ref[...]
    e = sae_ref[...] - x
    l2 = jnp.sum(e * e)
    mu = jnp.mean(x, axis=0, keepdims=True)
    d = x - mu
    tv = jnp.sum(d * d)
    tv = jnp.where(tv == 0.0, 1.0, tv)
    fvu_ref[...] = jnp.reshape(l2 / tv, (1, 1))


def _fvu(x, sae):
    return pl.pallas_call(
        _fvu_body,
        out_shape=jax.ShapeDtypeStruct((1, 1), jnp.float32),
    )(x, sae)


def kernel(x, W_enc, b_enc, W_dec, b_dec):
    be2 = b_enc.reshape(1, N_DICT)
    bd2 = b_dec.reshape(1, D_IN)
    splits = (N_TOK // 2, N_TOK // 2)
    parts = []
    start = 0
    for n_h in splits:
        xh = lax.slice_in_dim(x, start, start + n_h, axis=0)
        start += n_h
        pre, gmax = _encode(xh, W_enc, be2, bd2, n_h)
        gsel = _gsel(gmax, n_h)
        cand_vals, cand_cols = _cand_gather(gsel, pre, n_h)
        top_acts, top_idx = _topk(cand_vals, cand_cols, n_h)
        sae_out = _decode(top_idx, top_acts, W_dec, b_dec, n_h)
        parts.append((sae_out, top_acts, top_idx))
    sae_out = jnp.concatenate([p[0] for p in parts], axis=0)
    top_acts = jnp.concatenate([p[1] for p in parts], axis=0)
    top_idx = jnp.concatenate([p[2] for p in parts], axis=0)
    fvu = _fvu(x, sae_out).reshape(())
    zero = jnp.zeros((), x.dtype)
    return sae_out, top_acts, top_idx, fvu, zero, zero
